# baseline (speedup 1.0000x reference)
import functools

import jax
import jax.numpy as jnp
from jax import lax
from jax.experimental import pallas as pl
from jax.experimental.pallas import tpu as pltpu
from jax.experimental.pallas import tpu_sc as plsc

B = 4
S = 2048
D = 1024
L = 16

NC = 2
NS = 16
NW = NC * NS
S_PER_W = S // NW
CH = 16
NCHUNK = S_PER_W // CH
NU = B * NCHUNK
NB = 3


def _make_kernel():
    mesh = plsc.VectorSubcoreMesh(core_axis_name="c", subcore_axis_name="s")

    @functools.partial(
        pl.kernel,
        mesh=mesh,
        out_type=jax.ShapeDtypeStruct((B, S, D), jnp.float32),
        scratch_types=[
            pltpu.VMEM((B * S_PER_W,), jnp.int32),
            pltpu.VMEM((S_PER_W, D), jnp.float32),
            pltpu.VMEM((NB, CH, D), jnp.float32),
        ]
        + [pltpu.SemaphoreType.DMA] * (2 * NB + 1),
    )
    def emb_kernel(tok_hbm, tab_hbm, pos_hbm, out_hbm, idx_v, pos_v, rows_v, *sems):
        gsem, ssem, psem = sems[:NB], sems[NB : 2 * NB], sems[2 * NB]
        wid = lax.axis_index("s") * NC + lax.axis_index("c")
        s0 = wid * S_PER_W

        for b in range(B):
            pltpu.sync_copy(
                tok_hbm.at[b, pl.ds(s0, S_PER_W)],
                idx_v.at[pl.ds(b * S_PER_W, S_PER_W)],
            )
        pos_cp = pltpu.async_copy(pos_hbm.at[pl.ds(s0, S_PER_W)], pos_v, psem)

        def start_gather(u):
            return pltpu.async_copy(
                tab_hbm.at[idx_v.at[pl.ds(u * CH, CH)]],
                rows_v.at[u % NB],
                gsem[u % NB],
            )

        gathers = {u: start_gather(u) for u in range(NB - 1)}
        pos_cp.wait()
        stores = {}
        for u in range(NU):
            b, j = divmod(u, NCHUNK)
            nb = u % NB
            gathers.pop(u).wait()
            nxt = u + NB - 1
            if nxt < NU:
                if nxt % NB in stores:
                    stores.pop(nxt % NB).wait()
                gathers[nxt] = start_gather(nxt)

            G = 8
            RPB = 2
            NL = D // L

            def add_rows(i, _, j=j, nb=nb):
                r0 = i * RPB
                slots = [
                    (r0 + dr, pl.ds(l * L, L))
                    for dr in range(RPB)
                    for l in range(NL)
                ]
                vals = [pos_v[j * CH + r, sl] for r, sl in slots[:G]]
                for k, (r, sl) in enumerate(slots):
                    if k + G < len(slots):
                        rn, sln = slots[k + G]
                        vals.append(pos_v[j * CH + rn, sln])
                    plsc.addupdate(rows_v.at[nb, r, sl], vals[k])
                return 0

            lax.fori_loop(0, CH // RPB, add_rows, 0)

            stores[nb] = pltpu.async_copy(
                rows_v.at[nb],
                out_hbm.at[b, pl.ds(s0 + j * CH, CH)],
                ssem[nb],
            )
        for st in stores.values():
            st.wait()

    return emb_kernel


def kernel(tokens, token_embedding, position_embedding):
    emb = _make_kernel()
    return emb(tokens.astype(jnp.int32), token_embedding, position_embedding)


# speedup vs baseline: 1.0724x; 1.0724x over previous
import functools

import jax
import jax.numpy as jnp
from jax import lax
from jax.experimental import pallas as pl
from jax.experimental.pallas import tpu as pltpu
from jax.experimental.pallas import tpu_sc as plsc

B = 4
S = 2048
D = 1024
L = 16

NC = 2
NS = 16
NW = NC * NS
S_PER_W = S // NW
CH = 16
NCHUNK = S_PER_W // CH
NU = B * NCHUNK
NB = 3


def _make_kernel():
    mesh = plsc.VectorSubcoreMesh(core_axis_name="c", subcore_axis_name="s")

    @functools.partial(
        pl.kernel,
        mesh=mesh,
        out_type=jax.ShapeDtypeStruct((B, S, D), jnp.float32),
        scratch_types=[
            pltpu.VMEM((B * S_PER_W,), jnp.int32),
            pltpu.VMEM((S_PER_W, D), jnp.float32),
            pltpu.VMEM((NB, CH, D), jnp.float32),
        ]
        + [pltpu.SemaphoreType.DMA] * (2 * NB + 1),
    )
    def emb_kernel(tok_hbm, tab_hbm, pos_hbm, out_hbm, idx_v, pos_v, rows_v, *sems):
        gsem, ssem, psem = sems[:NB], sems[NB : 2 * NB], sems[2 * NB]
        wid = lax.axis_index("s") * NC + lax.axis_index("c")
        s0 = wid * S_PER_W

        for b in range(B):
            pltpu.sync_copy(
                tok_hbm.at[b, pl.ds(s0, S_PER_W)],
                idx_v.at[pl.ds(b * S_PER_W, S_PER_W)],
            )
        pos_cp = pltpu.async_copy(pos_hbm.at[pl.ds(s0, S_PER_W)], pos_v, psem)

        def start_gather(u):
            return pltpu.async_copy(
                tab_hbm.at[idx_v.at[pl.ds(u * CH, CH)]],
                rows_v.at[u % NB],
                gsem[u % NB],
            )

        gathers = {u: start_gather(u) for u in range(NB - 1)}
        pos_cp.wait()
        stores = {}
        for u in range(NU):
            b, j = divmod(u, NCHUNK)
            nb = u % NB
            gathers.pop(u).wait()
            nxt = u + NB - 1
            if nxt < NU:
                if nxt % NB in stores:
                    stores.pop(nxt % NB).wait()
                gathers[nxt] = start_gather(nxt)

            G = 8
            NL = D // L

            def add_row(r, _, j=j, nb=nb):
                vals = [pos_v[j * CH + r, pl.ds(l * L, L)] for l in range(G)]
                for l in range(NL):
                    if l + G < NL:
                        vals.append(pos_v[j * CH + r, pl.ds((l + G) * L, L)])
                    plsc.addupdate(rows_v.at[nb, r, pl.ds(l * L, L)], vals[l])
                return 0

            lax.fori_loop(0, CH, add_row, 0)

            stores[nb] = pltpu.async_copy(
                rows_v.at[nb],
                out_hbm.at[b, pl.ds(s0 + j * CH, CH)],
                ssem[nb],
            )
        for st in stores.values():
            st.wait()

    return emb_kernel


def kernel(tokens, token_embedding, position_embedding):
    emb = _make_kernel()
    return emb(tokens.astype(jnp.int32), token_embedding, position_embedding)
